# baseline (speedup 1.0000x reference)
import jax
import jax.numpy as jnp
from jax.experimental import pallas as pl

N_TOKENS = 9216
QUANT_DIM = 1024
OUTPUT_DIM = 256

_TB = 1152


def _body(x_ref, wh_ref, wl_ref, out_ref):
    xb = x_ref[...]
    idx = jnp.argmax(xb, axis=-1)
    iota = jax.lax.broadcasted_iota(jnp.int32, (_TB, QUANT_DIM), 1)
    oh = (iota == idx[:, None]).astype(jnp.bfloat16)
    dims = (((1,), (1,)), ((), ()))
    acc = jax.lax.dot_general(
        oh, wh_ref[...], dims, preferred_element_type=jnp.float32
    )
    acc += jax.lax.dot_general(
        oh, wl_ref[...], dims, preferred_element_type=jnp.float32
    )
    out_ref[...] = acc


def kernel(x, W):
    w_hi = W.astype(jnp.bfloat16)
    w_lo = (W - w_hi.astype(jnp.float32)).astype(jnp.bfloat16)
    grid = N_TOKENS // _TB
    return pl.pallas_call(
        _body,
        grid=(grid,),
        in_specs=[
            pl.BlockSpec((_TB, QUANT_DIM), lambda i: (i, 0)),
            pl.BlockSpec((OUTPUT_DIM, QUANT_DIM), lambda i: (0, 0)),
            pl.BlockSpec((OUTPUT_DIM, QUANT_DIM), lambda i: (0, 0)),
        ],
        out_specs=pl.BlockSpec((_TB, OUTPUT_DIM), lambda i: (i, 0)),
        out_shape=jax.ShapeDtypeStruct((N_TOKENS, OUTPUT_DIM), jnp.float32),
    )(x, w_hi, w_lo)


# speedup vs baseline: 1.0151x; 1.0151x over previous
import jax
import jax.numpy as jnp
from jax.experimental import pallas as pl

N_TOKENS = 9216
QUANT_DIM = 1024
OUTPUT_DIM = 256

_TB = 2304


def _body(x_ref, wh_ref, wl_ref, out_ref):
    xb = x_ref[...]
    idx = jnp.argmax(xb, axis=-1)
    iota = jax.lax.broadcasted_iota(jnp.int32, (_TB, QUANT_DIM), 1)
    oh = (iota == idx[:, None]).astype(jnp.bfloat16)
    dims = (((1,), (1,)), ((), ()))
    acc = jax.lax.dot_general(
        oh, wh_ref[...], dims, preferred_element_type=jnp.float32
    )
    acc += jax.lax.dot_general(
        oh, wl_ref[...], dims, preferred_element_type=jnp.float32
    )
    out_ref[...] = acc


def kernel(x, W):
    w_hi = W.astype(jnp.bfloat16)
    w_lo = (W - w_hi.astype(jnp.float32)).astype(jnp.bfloat16)
    grid = N_TOKENS // _TB
    return pl.pallas_call(
        _body,
        grid=(grid,),
        in_specs=[
            pl.BlockSpec((_TB, QUANT_DIM), lambda i: (i, 0)),
            pl.BlockSpec((OUTPUT_DIM, QUANT_DIM), lambda i: (0, 0)),
            pl.BlockSpec((OUTPUT_DIM, QUANT_DIM), lambda i: (0, 0)),
        ],
        out_specs=pl.BlockSpec((_TB, OUTPUT_DIM), lambda i: (i, 0)),
        out_shape=jax.ShapeDtypeStruct((N_TOKENS, OUTPUT_DIM), jnp.float32),
    )(x, w_hi, w_lo)
